# Initial kernel scaffold; baseline (speedup 1.0000x reference)
import jax
import jax.numpy as jnp
from jax.experimental import pallas as pl


def kernel(team_skill, conf_skill, team, conf):
    raise NotImplementedError("write your pallas kernel here")


# speedup vs baseline: 1.0578x; 1.0578x over previous
import jax
import jax.numpy as jnp
from jax import lax
from jax.experimental import pallas as pl
from jax.experimental.pallas import tpu as pltpu
from jax.experimental.pallas import tpu_sc as plsc

B = 16384
D = 64
NC, NS = 2, 16
NW = NC * NS
BPW = B // NW
CHUNK = 128
NCHUNK = BPW // CHUNK
LANES = 16


def _body(team_hbm, conf_hbm, tidx_hbm, cidx_hbm, out_hbm,
          tidx_v, cidx_v, trows, crows, sem):
    wid = lax.axis_index("s") * NC + lax.axis_index("c")
    rowblk = wid * NCHUNK
    pltpu.sync_copy(tidx_hbm.at[pl.ds(rowblk, NCHUNK)], tidx_v)
    pltpu.sync_copy(cidx_hbm.at[pl.ds(rowblk, NCHUNK)], cidx_v)
    copies = []
    for j in range(NCHUNK):
        dst = pl.ds(j * CHUNK, CHUNK)
        copies.append(pltpu.async_copy(team_hbm.at[tidx_v.at[j]], trows.at[dst], sem))
        copies.append(pltpu.async_copy(conf_hbm.at[cidx_v.at[j]], crows.at[dst], sem))
    for cp in copies:
        cp.wait()

    def add_row(r, carry):
        for c in range(D // LANES):
            sl = pl.ds(c * LANES, LANES)
            trows[r, sl] = trows[r, sl] + crows[r, sl]
        return carry

    lax.fori_loop(0, BPW, add_row, 0)
    pltpu.sync_copy(trows, out_hbm.at[pl.ds(wid * BPW, BPW)])


def kernel(team_skill, conf_skill, team, conf):
    tidx = team.reshape(-1).astype(jnp.int32).reshape(B // CHUNK, CHUNK)
    cidx = conf.reshape(-1).astype(jnp.int32).reshape(B // CHUNK, CHUNK)
    mesh = plsc.VectorSubcoreMesh(
        core_axis_name="c", subcore_axis_name="s",
        num_cores=NC, num_subcores=NS)
    f = pl.kernel(
        _body,
        out_type=jax.ShapeDtypeStruct((B, D), jnp.float32),
        mesh=mesh,
        scratch_types=[
            pltpu.VMEM((NCHUNK, CHUNK), jnp.int32),
            pltpu.VMEM((NCHUNK, CHUNK), jnp.int32),
            pltpu.VMEM((BPW, D), jnp.float32),
            pltpu.VMEM((BPW, D), jnp.float32),
            pltpu.SemaphoreType.DMA,
        ],
        compiler_params=pltpu.CompilerParams(use_tc_tiling_on_sc=False),
    )
    return f(team_skill, conf_skill, tidx, cidx)


# speedup vs baseline: 1.5265x; 1.4430x over previous
import jax
import jax.numpy as jnp
from jax import lax
from jax.experimental import pallas as pl
from jax.experimental.pallas import tpu as pltpu
from jax.experimental.pallas import tpu_sc as plsc

B = 16384
D = 64
VT = 100000
VC = 1000
NC, NS = 2, 16
NW = NC * NS
FPW = D // NW
CHUNK = 2048
NCHUNK = B // CHUNK
LANES = 16


def _body(pt_hbm, pc_hbm, tidx_hbm, cidx_hbm, out_hbm,
          row_v, crow_v, tidx_v, cidx_v, outc_v, sem):
    wid = lax.axis_index("s") * NC + lax.axis_index("c")
    for j in range(FPW):
        f = wid * FPW + j
        pltpu.sync_copy(pt_hbm.at[f], row_v)
        pltpu.sync_copy(pc_hbm.at[f], crow_v)
        for ch in range(NCHUNK):
            sl_b = pl.ds(ch * CHUNK, CHUNK)
            pltpu.sync_copy(tidx_hbm.at[sl_b], tidx_v)
            pltpu.sync_copy(cidx_hbm.at[sl_b], cidx_v)

            def gbody(i, carry):
                sl = pl.ds(i * LANES, LANES)
                tv = plsc.load_gather(row_v, [tidx_v[sl]])
                cv = plsc.load_gather(crow_v, [cidx_v[sl]])
                outc_v[sl] = tv + cv
                return carry

            lax.fori_loop(0, CHUNK // LANES, gbody, 0)
            pltpu.sync_copy(outc_v, out_hbm.at[f, sl_b])


def kernel(team_skill, conf_skill, team, conf):
    pt = team_skill.T
    pc = conf_skill.T
    tidx = team.reshape(-1).astype(jnp.int32)
    cidx = conf.reshape(-1).astype(jnp.int32)
    mesh = plsc.VectorSubcoreMesh(
        core_axis_name="c", subcore_axis_name="s",
        num_cores=NC, num_subcores=NS)
    f = pl.kernel(
        _body,
        out_type=jax.ShapeDtypeStruct((D, B), jnp.float32),
        mesh=mesh,
        scratch_types=[
            pltpu.VMEM((VT,), jnp.float32),
            pltpu.VMEM((VC,), jnp.float32),
            pltpu.VMEM((CHUNK,), jnp.int32),
            pltpu.VMEM((CHUNK,), jnp.int32),
            pltpu.VMEM((CHUNK,), jnp.float32),
            pltpu.SemaphoreType.DMA,
        ],
        compiler_params=pltpu.CompilerParams(
            use_tc_tiling_on_sc=True, needs_layout_passes=False),
    )
    out_t = f(pt, pc, tidx, cidx)
    return out_t.T


# speedup vs baseline: 1.8944x; 1.2410x over previous
import jax
import jax.numpy as jnp
from jax import lax
from jax.experimental import pallas as pl
from jax.experimental.pallas import tpu as pltpu
from jax.experimental.pallas import tpu_sc as plsc

B = 16384
D = 64
VT = 100000
VC = 1000
NC, NS = 2, 16
NW = NC * NS
FPW = D // NW
CHUNK = 2048
NCHUNK = B // CHUNK
LANES = 16


UNROLL = 8


def _body(pt_hbm, pc_hbm, tidx_hbm, cidx_hbm, out_hbm,
          row_v, crow_v, tidx_v, cidx_v, outc_v, sem_in, sem_out):
    wid = lax.axis_index("s") * NC + lax.axis_index("c")
    for j in range(FPW):
        f = wid * FPW + j
        row_cp = pltpu.async_copy(pt_hbm.at[f], row_v, sem_in)
        crow_cp = pltpu.async_copy(pc_hbm.at[f], crow_v, sem_in)
        idx_cps = [None] * NCHUNK
        out_cps = [None] * NCHUNK
        idx_cps[0] = (
            pltpu.async_copy(tidx_hbm.at[pl.ds(0, CHUNK)], tidx_v.at[0], sem_in),
            pltpu.async_copy(cidx_hbm.at[pl.ds(0, CHUNK)], cidx_v.at[0], sem_in),
        )
        row_cp.wait()
        crow_cp.wait()
        for ch in range(NCHUNK):
            s = ch % 2
            idx_cps[ch][0].wait()
            idx_cps[ch][1].wait()
            if ch + 1 < NCHUNK:
                nsl = pl.ds((ch + 1) * CHUNK, CHUNK)
                idx_cps[ch + 1] = (
                    pltpu.async_copy(tidx_hbm.at[nsl], tidx_v.at[1 - s], sem_in),
                    pltpu.async_copy(cidx_hbm.at[nsl], cidx_v.at[1 - s], sem_in),
                )
            if ch >= 2:
                out_cps[ch - 2].wait()

            def gbody(i, carry):
                for u in range(UNROLL):
                    sl = pl.ds((i * UNROLL + u) * LANES, LANES)
                    tv = plsc.load_gather(row_v, [tidx_v[s, sl]])
                    cv = plsc.load_gather(crow_v, [cidx_v[s, sl]])
                    outc_v[s, sl] = tv + cv
                return carry

            lax.fori_loop(0, CHUNK // (LANES * UNROLL), gbody, 0)
            out_cps[ch] = pltpu.async_copy(
                outc_v.at[s], out_hbm.at[f, pl.ds(ch * CHUNK, CHUNK)], sem_out)
        out_cps[NCHUNK - 2].wait()
        out_cps[NCHUNK - 1].wait()


def kernel(team_skill, conf_skill, team, conf):
    pt = team_skill.T
    pc = conf_skill.T
    tidx = team.reshape(-1).astype(jnp.int32)
    cidx = conf.reshape(-1).astype(jnp.int32)
    mesh = plsc.VectorSubcoreMesh(
        core_axis_name="c", subcore_axis_name="s",
        num_cores=NC, num_subcores=NS)
    f = pl.kernel(
        _body,
        out_type=jax.ShapeDtypeStruct((D, B), jnp.float32),
        mesh=mesh,
        scratch_types=[
            pltpu.VMEM((VT,), jnp.float32),
            pltpu.VMEM((VC,), jnp.float32),
            pltpu.VMEM((2, CHUNK), jnp.int32),
            pltpu.VMEM((2, CHUNK), jnp.int32),
            pltpu.VMEM((2, CHUNK), jnp.float32),
            pltpu.SemaphoreType.DMA,
            pltpu.SemaphoreType.DMA,
        ],
        compiler_params=pltpu.CompilerParams(
            use_tc_tiling_on_sc=True, needs_layout_passes=False),
    )
    out_t = f(pt, pc, tidx, cidx)
    return out_t.T


# speedup vs baseline: 2.0311x; 1.0722x over previous
import jax
import jax.numpy as jnp
from jax import lax
from jax.experimental import pallas as pl
from jax.experimental.pallas import tpu as pltpu
from jax.experimental.pallas import tpu_sc as plsc

B = 16384
D = 64
VT = 100000
VC = 1000
NC, NS = 2, 16
NW = NC * NS
FPW = D // NW
CHUNK = 2048
NCHUNK = B // CHUNK
LANES = 16


UNROLL = 8


def _body(pt_hbm, pc_hbm, tidx_hbm, cidx_hbm, out_hbm,
          row_v, crow_v, tidx_v, cidx_v, outc_v, sem_in, sem_out):
    wid = lax.axis_index("s") * NC + lax.axis_index("c")
    for j in range(FPW):
        f = wid * FPW + j
        row_cp = pltpu.async_copy(pt_hbm.at[f], row_v, sem_in)
        crow_cp = pltpu.async_copy(pc_hbm.at[f], crow_v, sem_in)
        idx_cps = [None] * NCHUNK
        out_cps = [None] * NCHUNK
        idx_cps[0] = (
            pltpu.async_copy(tidx_hbm.at[pl.ds(0, CHUNK)], tidx_v.at[0], sem_in),
            pltpu.async_copy(cidx_hbm.at[pl.ds(0, CHUNK)], cidx_v.at[0], sem_in),
        )
        row_cp.wait()
        crow_cp.wait()
        for ch in range(NCHUNK):
            s = ch % 2
            idx_cps[ch][0].wait()
            idx_cps[ch][1].wait()
            if ch + 1 < NCHUNK:
                nsl = pl.ds((ch + 1) * CHUNK, CHUNK)
                idx_cps[ch + 1] = (
                    pltpu.async_copy(tidx_hbm.at[nsl], tidx_v.at[1 - s], sem_in),
                    pltpu.async_copy(cidx_hbm.at[nsl], cidx_v.at[1 - s], sem_in),
                )
            if ch >= 2:
                out_cps[ch - 2].wait()

            @plsc.parallel_loop(0, CHUNK // LANES, step=1, unroll=UNROLL)
            def gloop(i):
                sl = pl.ds(i * LANES, LANES)
                tv = plsc.load_gather(row_v, [tidx_v[s, sl]])
                cv = plsc.load_gather(crow_v, [cidx_v[s, sl]])
                outc_v[s, sl] = tv + cv
            out_cps[ch] = pltpu.async_copy(
                outc_v.at[s], out_hbm.at[f, pl.ds(ch * CHUNK, CHUNK)], sem_out)
        out_cps[NCHUNK - 2].wait()
        out_cps[NCHUNK - 1].wait()


def kernel(team_skill, conf_skill, team, conf):
    pt = team_skill.T
    pc = conf_skill.T
    tidx = team.reshape(-1).astype(jnp.int32)
    cidx = conf.reshape(-1).astype(jnp.int32)
    mesh = plsc.VectorSubcoreMesh(
        core_axis_name="c", subcore_axis_name="s",
        num_cores=NC, num_subcores=NS)
    f = pl.kernel(
        _body,
        out_type=jax.ShapeDtypeStruct((D, B), jnp.float32),
        mesh=mesh,
        scratch_types=[
            pltpu.VMEM((VT,), jnp.float32),
            pltpu.VMEM((VC,), jnp.float32),
            pltpu.VMEM((2, CHUNK), jnp.int32),
            pltpu.VMEM((2, CHUNK), jnp.int32),
            pltpu.VMEM((2, CHUNK), jnp.float32),
            pltpu.SemaphoreType.DMA,
            pltpu.SemaphoreType.DMA,
        ],
        compiler_params=pltpu.CompilerParams(
            use_tc_tiling_on_sc=True, needs_layout_passes=False),
    )
    out_t = f(pt, pc, tidx, cidx)
    return out_t.T


# speedup vs baseline: 2.2672x; 1.1162x over previous
import jax
import jax.numpy as jnp
from jax import lax
from jax.experimental import pallas as pl
from jax.experimental.pallas import tpu as pltpu
from jax.experimental.pallas import tpu_sc as plsc

B = 16384
D = 64
VT = 100000
VC = 1000
NC, NS = 2, 16
NW = NC * NS
FPW = D // NW
CHUNK = 2048
NCHUNK = B // CHUNK
LANES = 16


UNROLL = 8


def _body(pt_hbm, pc_hbm, pidx_hbm, out_hbm,
          row_v, crow_v, pidx_v, outc_v, sem_in, sem_out):
    wid = lax.axis_index("s") * NC + lax.axis_index("c")
    tmask = jnp.int32((1 << 17) - 1)
    for j in range(FPW):
        f = wid * FPW + j
        row_cp = pltpu.async_copy(pt_hbm.at[f], row_v, sem_in)
        crow_cp = pltpu.async_copy(pc_hbm.at[f], crow_v, sem_in)
        idx_cps = [None] * NCHUNK
        out_cps = [None] * NCHUNK
        idx_cps[0] = pltpu.async_copy(
            pidx_hbm.at[pl.ds(0, CHUNK)], pidx_v.at[0], sem_in)
        row_cp.wait()
        crow_cp.wait()
        for ch in range(NCHUNK):
            s = ch % 2
            idx_cps[ch].wait()
            if ch + 1 < NCHUNK:
                nsl = pl.ds((ch + 1) * CHUNK, CHUNK)
                idx_cps[ch + 1] = pltpu.async_copy(
                    pidx_hbm.at[nsl], pidx_v.at[1 - s], sem_in)
            if ch >= 2:
                out_cps[ch - 2].wait()

            @plsc.parallel_loop(0, CHUNK // LANES, step=1, unroll=UNROLL)
            def gloop(i):
                sl = pl.ds(i * LANES, LANES)
                pk = pidx_v[s, sl]
                tv = plsc.load_gather(row_v, [pk & tmask])
                cv = plsc.load_gather(
                    crow_v, [lax.shift_right_logical(pk, 17)])
                outc_v[s, sl] = tv + cv
            out_cps[ch] = pltpu.async_copy(
                outc_v.at[s], out_hbm.at[f, pl.ds(ch * CHUNK, CHUNK)], sem_out)
        out_cps[NCHUNK - 2].wait()
        out_cps[NCHUNK - 1].wait()


def kernel(team_skill, conf_skill, team, conf):
    pt = team_skill.T
    pc = conf_skill.T
    tidx = team.reshape(-1).astype(jnp.int32)
    cidx = conf.reshape(-1).astype(jnp.int32)
    pidx = tidx | (cidx << 17)
    mesh = plsc.VectorSubcoreMesh(
        core_axis_name="c", subcore_axis_name="s",
        num_cores=NC, num_subcores=NS)
    f = pl.kernel(
        _body,
        out_type=jax.ShapeDtypeStruct((D, B), jnp.float32),
        mesh=mesh,
        scratch_types=[
            pltpu.VMEM((VT,), jnp.float32),
            pltpu.VMEM((VC,), jnp.float32),
            pltpu.VMEM((2, CHUNK), jnp.int32),
            pltpu.VMEM((2, CHUNK), jnp.float32),
            pltpu.SemaphoreType.DMA,
            pltpu.SemaphoreType.DMA,
        ],
        compiler_params=pltpu.CompilerParams(
            use_tc_tiling_on_sc=True, needs_layout_passes=False),
    )
    out_t = f(pt, pc, pidx)
    return out_t.T


# speedup vs baseline: 2.6520x; 1.1698x over previous
import jax
import jax.numpy as jnp
from jax import lax
from jax.experimental import pallas as pl
from jax.experimental.pallas import tpu as pltpu
from jax.experimental.pallas import tpu_sc as plsc

B = 16384
D = 64
VT = 100000
VC = 1000
NC, NS = 2, 16
NW = NC * NS
FPW = D // NW
CHUNK = 4096
NCHUNK = B // CHUNK
LANES = 16


UNROLL = 8


def _body(pt_hbm, pc_hbm, pidx_hbm, out_hbm,
          row_v, crow_v, pidx_v, outc_v, sem_in, sem_out):
    wid = lax.axis_index("s") * NC + lax.axis_index("c")
    tmask = jnp.int32((1 << 17) - 1)
    ngl = FPW * NCHUNK
    idx_cps = {0: pltpu.async_copy(
        pidx_hbm.at[pl.ds(0, CHUNK)], pidx_v.at[0], sem_in)}
    out_pending = []
    for j in range(FPW):
        f = wid * FPW + j
        row_cp = pltpu.async_copy(pt_hbm.at[f], row_v, sem_in)
        crow_cp = pltpu.async_copy(pc_hbm.at[f], crow_v, sem_in)
        row_cp.wait()
        crow_cp.wait()
        for ch in range(NCHUNK):
            g = j * NCHUNK + ch
            s = g % 2
            idx_cps.pop(g).wait()
            if g + 1 < ngl:
                nsl = pl.ds(((g + 1) % NCHUNK) * CHUNK, CHUNK)
                idx_cps[g + 1] = pltpu.async_copy(
                    pidx_hbm.at[nsl], pidx_v.at[1 - s], sem_in)
            while len(out_pending) >= 2:
                out_pending.pop(0).wait()

            @plsc.parallel_loop(0, CHUNK // LANES, step=1, unroll=UNROLL)
            def gloop(i):
                sl = pl.ds(i * LANES, LANES)
                pk = pidx_v[s, sl]
                tv = plsc.load_gather(row_v, [pk & tmask])
                cv = plsc.load_gather(
                    crow_v, [lax.shift_right_logical(pk, 17)])
                outc_v[s, sl] = tv + cv
            out_pending.append(pltpu.async_copy(
                outc_v.at[s], out_hbm.at[f, pl.ds(ch * CHUNK, CHUNK)], sem_out))
    for cp in out_pending:
        cp.wait()


def kernel(team_skill, conf_skill, team, conf):
    pt = team_skill.T
    pc = conf_skill.T
    tidx = team.reshape(-1).astype(jnp.int32)
    cidx = conf.reshape(-1).astype(jnp.int32)
    pidx = tidx | (cidx << 17)
    mesh = plsc.VectorSubcoreMesh(
        core_axis_name="c", subcore_axis_name="s",
        num_cores=NC, num_subcores=NS)
    f = pl.kernel(
        _body,
        out_type=jax.ShapeDtypeStruct((D, B), jnp.float32),
        mesh=mesh,
        scratch_types=[
            pltpu.VMEM((VT,), jnp.float32),
            pltpu.VMEM((VC,), jnp.float32),
            pltpu.VMEM((2, CHUNK), jnp.int32),
            pltpu.VMEM((2, CHUNK), jnp.float32),
            pltpu.SemaphoreType.DMA,
            pltpu.SemaphoreType.DMA,
        ],
        compiler_params=pltpu.CompilerParams(
            use_tc_tiling_on_sc=True, needs_layout_passes=False),
    )
    out_t = f(pt, pc, pidx)
    return out_t.T
